# baseline (speedup 1.0000x reference)
import functools

import jax
import jax.numpy as jnp
from jax import lax
from jax.experimental import pallas as pl
from jax.experimental.pallas import tpu as pltpu
from jax.experimental.pallas import tpu_sc as plsc

_NBUF = 4


def _pick_chunk(n_rows_per_batch: int, d: int, nb: int) -> int:
    budget = 460_000 // nb
    best = 0
    for ch in range(16, n_rows_per_batch + 1, 16):
        if n_rows_per_batch % ch:
            continue
        if ch * d * 4 + ch * 4 <= budget:
            best = ch
    if best == 0:
        raise ValueError("no valid chunk size")
    return best


@functools.partial(jax.jit, static_argnames=("bsz", "n", "d", "ch", "nb"))
def _sc_gather(table, idx, bsz, n, d, ch, nb):
    nchunks = (bsz * n) // ch
    chunks_per_batch = n // ch
    mesh = plsc.VectorSubcoreMesh(core_axis_name="c", subcore_axis_name="s")
    info = plsc.get_sparse_core_info()
    nw = info.num_cores * info.num_subcores

    @functools.partial(
        pl.kernel,
        out_type=jax.ShapeDtypeStruct((bsz * n, d), table.dtype),
        mesh=mesh,
        scratch_types=(
            [pltpu.VMEM((ch,), jnp.int32) for _ in range(nb)]
            + [pltpu.VMEM((ch, d), table.dtype) for _ in range(nb)]
            + [pltpu.SemaphoreType.DMA for _ in range(3 * nb)]
        ),
    )
    def k(table_hbm, idx_hbm, out_hbm, *scr):
        idxs, rows = scr[0:nb], scr[nb:2 * nb]
        semi, semg, sems = scr[2 * nb:3 * nb], scr[3 * nb:4 * nb], scr[4 * nb:]
        bufs = tuple(zip(idxs, rows, semi, semg, sems))
        wid = lax.axis_index("s") * info.num_cores + lax.axis_index("c")
        iters = (nchunks + nw - 1) // nw

        def drain_scatter(rows_v, sem):
            pltpu.make_async_copy(rows_v, out_hbm.at[pl.ds(0, ch)], sem).wait()

        def idx_src(c):
            return idx_hbm.at[pl.ds(c * ch, ch)]

        for par, (idx_v, _, si, _, _) in enumerate(bufs):
            c0 = wid + par * nw

            @pl.when(c0 < nchunks)
            def _():
                pltpu.async_copy(idx_src(c0), idx_v, si)

        kmax = iters + 1
        kmax_r = kmax + (-kmax) % nb

        @pl.loop(0, kmax_r, step=nb)
        def _(i):
            for par in range(nb):
                k_it = i + par
                c = wid + k_it * nw
                idx_v, rows_v, si, sg, ss = bufs[par]
                pidx_v, prows_v, psi, psg, pss = bufs[(par - 1) % nb]

                @pl.when(c < nchunks)
                def _():
                    pltpu.make_async_copy(
                        idx_hbm.at[pl.ds(0, ch)], idx_v, si).wait()
                    boff = (c // chunks_per_batch) * n

                    @pl.loop(0, ch, step=16)
                    def _(j):
                        sl = pl.ds(j, 16)
                        idx_v[sl] = idx_v[sl] + boff

                    @pl.when(k_it >= nb)
                    def _():
                        drain_scatter(rows_v, ss)

                    pltpu.async_copy(table_hbm.at[idx_v], rows_v, sg)

                @pl.when((k_it >= 1) & (c - nw < nchunks))
                def _():
                    pltpu.make_async_copy(
                        table_hbm.at[pidx_v], prows_v, psg).wait()
                    pltpu.async_copy(
                        prows_v, out_hbm.at[pl.ds((c - nw) * ch, ch)], pss)

                    @pl.when(c + (nb - 1) * nw < nchunks)
                    def _():
                        pltpu.async_copy(
                            idx_src(c + (nb - 1) * nw), pidx_v, psi)

        for par, (_, rows_v, _, _, ss) in enumerate(bufs):
            @pl.when(wid + par * nw < nchunks)
            def _():
                drain_scatter(rows_v, ss)

    return k(table, idx)


def kernel(vertices, reverse_map):
    bsz, n, d = vertices.shape
    ch = _pick_chunk(n, d, _NBUF)
    table = vertices.reshape(bsz * n, d)
    idx = reverse_map.reshape(bsz * n).astype(jnp.int32)
    out = _sc_gather(table, idx, bsz, n, d, ch, _NBUF)
    return out.reshape(bsz, n, d)
